# baseline (speedup 1.0000x reference)
import math

import jax
import jax.numpy as jnp
from jax.experimental import pallas as pl
from jax.experimental.pallas import tpu as pltpu

_V = 100000
_PADDING_IDX = 1
_SMOOTHING = 0.1
_CONF = 1.0 - _SMOOTHING
_EPS = _SMOOTHING / _V
_C1 = (_V - 1) * _EPS * math.log(_EPS) + _CONF * math.log(_CONF)

_N = 512
_RB = 64
_NR = _N // _RB
_VB = 4096
_NB = -(-_V // _VB)
_NK = _VB // 128


def _stats_body(x_ref, t_ref, kl_ref, m_sc, s_sc, sx_sc, xt_sc):
    j = pl.program_id(1)
    lane = jax.lax.broadcasted_iota(jnp.int32, (_RB, 128), 1)
    t_loc = t_ref[...] - j * _VB

    @pl.when(j == 0)
    def _init():
        m_sc[...] = jnp.full((_RB, 128), -jnp.inf, jnp.float32)
        s_sc[...] = jnp.zeros((_RB, 128), jnp.float32)
        sx_sc[...] = jnp.zeros((_RB, 128), jnp.float32)
        xt_sc[...] = jnp.zeros((_RB, 128), jnp.float32)

    def _accumulate(masked_tail):
        mb = jnp.full((_RB, 128), -jnp.inf, jnp.float32)
        sxa = sx_sc[...]
        xta = xt_sc[...]
        nvalid = _V - j * _VB
        for k in range(_NK):
            xc = x_ref[:, k * 128:(k + 1) * 128]
            if masked_tail:
                ok = (k * 128 + lane) < nvalid
                xc = jnp.where(ok, xc, 0.0)
                mb = jnp.maximum(mb, jnp.where(ok, xc, -jnp.inf))
            else:
                mb = jnp.maximum(mb, xc)
            sxa = sxa + xc
            xta = xta + jnp.where(lane == t_loc - k * 128, xc, 0.0)
        sx_sc[...] = sxa
        xt_sc[...] = xta
        m0 = m_sc[...]
        mn = jnp.maximum(m0, mb)
        s = s_sc[...] * jnp.exp(m0 - mn)
        for k in range(_NK):
            xc = x_ref[:, k * 128:(k + 1) * 128]
            e = jnp.exp(xc - mn)
            if masked_tail:
                e = jnp.where((k * 128 + lane) < nvalid, e, 0.0)
            s = s + e
        m_sc[...] = mn
        s_sc[...] = s
        return mn, s

    @pl.when(j < _NB - 1)
    def _full():
        _accumulate(False)

    @pl.when(j == _NB - 1)
    def _tail_and_reduce():
        mn, s = _accumulate(True)
        mrow = jnp.max(mn, axis=1, keepdims=True)
        srow = jnp.sum(s * jnp.exp(mn - mrow), axis=1, keepdims=True)
        logz = mrow + jnp.log(srow)
        sxr = jnp.sum(sx_sc[...], axis=1, keepdims=True)
        xtr = jnp.sum(xt_sc[...], axis=1, keepdims=True)
        row_kl = (jnp.float32(_C1)
                  - jnp.float32(_EPS) * (sxr - jnp.float32(_V) * logz)
                  - jnp.float32(_CONF - _EPS) * (xtr - logz))
        row_kl = jnp.where(t_ref[...] == _PADDING_IDX, 0.0, row_kl)
        kl_ref[0] = jnp.sum(row_kl, axis=0, keepdims=True)


def _finalize_body(kl_ref, dl_ref, out_ref):
    denom = jnp.sum(dl_ref[...], axis=0, keepdims=True)
    out_ref[...] = jnp.sum(kl_ref[...], axis=0, keepdims=True) / denom


def kernel(pred, targets, decode_lengths):
    x = pred.reshape(_N, _V)
    t = targets.reshape(_N, 1).astype(jnp.int32)
    dl = decode_lengths.reshape(-1, 1).astype(jnp.float32)

    klp = pl.pallas_call(
        _stats_body,
        grid=(_NR, _NB),
        in_specs=[
            pl.BlockSpec((_RB, _VB), lambda r, j: (r, j)),
            pl.BlockSpec((_RB, 1), lambda r, j: (r, 0)),
        ],
        out_specs=pl.BlockSpec((1, 1, 1), lambda r, j: (r, 0, 0)),
        out_shape=jax.ShapeDtypeStruct((_NR, 1, 1), jnp.float32),
        scratch_shapes=[
            pltpu.VMEM((_RB, 128), jnp.float32),
            pltpu.VMEM((_RB, 128), jnp.float32),
            pltpu.VMEM((_RB, 128), jnp.float32),
            pltpu.VMEM((_RB, 128), jnp.float32),
        ],
        compiler_params=pltpu.CompilerParams(
            dimension_semantics=("arbitrary", "arbitrary"),
        ),
    )(x, t)

    out = pl.pallas_call(
        _finalize_body,
        out_shape=jax.ShapeDtypeStruct((1, 1), jnp.float32),
    )(klp.reshape(_NR, 1), dl)
    return out.reshape(())


# speedup vs baseline: 1.7973x; 1.7973x over previous
import math

import jax
import jax.numpy as jnp
from jax.experimental import pallas as pl
from jax.experimental.pallas import tpu as pltpu

_V = 100000
_PADDING_IDX = 1
_SMOOTHING = 0.1
_CONF = 1.0 - _SMOOTHING
_EPS = _SMOOTHING / _V
_C1 = (_V - 1) * _EPS * math.log(_EPS) + _CONF * math.log(_CONF)

_N = 512
_VB = 4096
_NB = -(-_V // _VB)


def _stats_body(x_ref, t_ref, dl_ref, out_ref, m_sc, s_sc, sx_sc, xt_sc):
    j = pl.program_id(0)

    @pl.when(j == 0)
    def _init():
        m_sc[...] = jnp.full((_N, 1), -jnp.inf, jnp.float32)
        s_sc[...] = jnp.zeros((_N, 1), jnp.float32)
        sx_sc[...] = jnp.zeros((_N, 1), jnp.float32)
        xt_sc[...] = jnp.zeros((_N, 1), jnp.float32)

    x = x_ref[...]
    t_loc = t_ref[...] - j * _VB
    lane = jax.lax.broadcasted_iota(jnp.int32, (_N, _VB), 1)

    @pl.when(j < _NB - 1)
    def _full():
        m0 = m_sc[...]
        mn = jnp.maximum(m0, jnp.max(x, axis=1, keepdims=True))
        s_sc[...] = (s_sc[...] * jnp.exp(m0 - mn)
                     + jnp.sum(jnp.exp(x - mn), axis=1, keepdims=True))
        m_sc[...] = mn
        sx_sc[...] += jnp.sum(x, axis=1, keepdims=True)
        xt_sc[...] += jnp.sum(jnp.where(lane == t_loc, x, 0.0),
                              axis=1, keepdims=True)

    @pl.when(j == _NB - 1)
    def _tail_and_finalize():
        valid = lane < (_V - j * _VB)
        xm = jnp.where(valid, x, -jnp.inf)
        x0 = jnp.where(valid, x, 0.0)
        m0 = m_sc[...]
        mn = jnp.maximum(m0, jnp.max(xm, axis=1, keepdims=True))
        s = (s_sc[...] * jnp.exp(m0 - mn)
             + jnp.sum(jnp.exp(xm - mn), axis=1, keepdims=True))
        sx = sx_sc[...] + jnp.sum(x0, axis=1, keepdims=True)
        xt = xt_sc[...] + jnp.sum(jnp.where(lane == t_loc, x0, 0.0),
                                  axis=1, keepdims=True)
        logz = mn + jnp.log(s)
        row_kl = (jnp.float32(_C1)
                  - jnp.float32(_EPS) * (sx - jnp.float32(_V) * logz)
                  - jnp.float32(_CONF - _EPS) * (xt - logz))
        row_kl = jnp.where(t_ref[...] == _PADDING_IDX, 0.0, row_kl)
        denom = jnp.sum(dl_ref[...], axis=0, keepdims=True)
        out_ref[...] = jnp.sum(row_kl, axis=0, keepdims=True) / denom


def kernel(pred, targets, decode_lengths):
    x = pred.reshape(_N, _V)
    t = targets.reshape(_N, 1).astype(jnp.int32)
    dl = decode_lengths.reshape(-1, 1).astype(jnp.float32)

    out = pl.pallas_call(
        _stats_body,
        grid=(_NB,),
        in_specs=[
            pl.BlockSpec((_N, _VB), lambda j: (0, j)),
            pl.BlockSpec((_N, 1), lambda j: (0, 0)),
            pl.BlockSpec((dl.shape[0], 1), lambda j: (0, 0)),
        ],
        out_specs=pl.BlockSpec((1, 1), lambda j: (0, 0)),
        out_shape=jax.ShapeDtypeStruct((1, 1), jnp.float32),
        scratch_shapes=[
            pltpu.VMEM((_N, 1), jnp.float32),
            pltpu.VMEM((_N, 1), jnp.float32),
            pltpu.VMEM((_N, 1), jnp.float32),
            pltpu.VMEM((_N, 1), jnp.float32),
        ],
        compiler_params=pltpu.CompilerParams(
            dimension_semantics=("arbitrary",),
        ),
    )(x, t, dl)
    return out.reshape(())
